# Initial kernel scaffold; baseline (speedup 1.0000x reference)
import jax
import jax.numpy as jnp
from jax.experimental import pallas as pl


def kernel(text_batch, table, W, b):
    raise NotImplementedError("write your pallas kernel here")


# speedup vs baseline: 3.3087x; 3.3087x over previous
import functools

import jax
import jax.numpy as jnp
from jax import lax
from jax.experimental import pallas as pl
from jax.experimental.pallas import tpu as pltpu
from jax.experimental.pallas import tpu_sc as plsc

_NUM_CORES = 2
_NUM_SUBCORES = 16
_NW = _NUM_CORES * _NUM_SUBCORES
_LANES = 16


def _make_sc_pool(B, L, EMB, V):
    assert B % _NW == 0 and EMB % _LANES == 0
    bpw = B // _NW
    inv_l = 1.0 / float(L)
    mesh = plsc.VectorSubcoreMesh(core_axis_name="c", subcore_axis_name="s")

    @functools.partial(
        pl.kernel,
        out_type=jax.ShapeDtypeStruct((B, EMB), jnp.float32),
        mesh=mesh,
        scratch_types=[
            pltpu.VMEM((bpw, L), jnp.int32),
            pltpu.VMEM((L, EMB), jnp.float32),
            pltpu.VMEM((L, EMB), jnp.float32),
            pltpu.VMEM((bpw, EMB), jnp.float32),
            pltpu.SemaphoreType.DMA,
            pltpu.SemaphoreType.DMA,
        ],
    )
    def sc_pool(text_hbm, table_hbm, out_hbm, idx_v, buf0, buf1, out_v, sem0, sem1):
        wid = lax.axis_index("c") * _NUM_SUBCORES + lax.axis_index("s")
        base = wid * bpw
        pltpu.sync_copy(text_hbm.at[pl.ds(base, bpw)], idx_v)

        def accumulate(buf, row):
            accs = [buf[0, pl.ds(cb * _LANES, _LANES)] for cb in range(EMB // _LANES)]
            for r in range(1, L):
                for cb in range(EMB // _LANES):
                    accs[cb] = accs[cb] + buf[r, pl.ds(cb * _LANES, _LANES)]
            for cb in range(EMB // _LANES):
                out_v[row, pl.ds(cb * _LANES, _LANES)] = accs[cb] * inv_l

        pltpu.async_copy(table_hbm.at[idx_v.at[0]], buf0, sem0)

        @pl.loop(0, bpw, step=2)
        def _(j):
            d1 = pltpu.async_copy(table_hbm.at[idx_v.at[j + 1]], buf1, sem1)
            pltpu.make_async_copy(table_hbm.at[idx_v.at[j]], buf0, sem0).wait()
            accumulate(buf0, j)

            @pl.when(j + 2 < bpw)
            def _():
                pltpu.async_copy(table_hbm.at[idx_v.at[j + 2]], buf0, sem0)

            d1.wait()
            accumulate(buf1, j + 1)

        pltpu.sync_copy(out_v, out_hbm.at[pl.ds(base, bpw)])

    return sc_pool


def _mm_body(p_ref, w_ref, b_ref, o_ref):
    o_ref[...] = (
        lax.dot_general(
            p_ref[...],
            w_ref[...],
            (((1,), (1,)), ((), ())),
            preferred_element_type=jnp.float32,
        )
        + b_ref[...]
    )


def _make_tc_matmul(B, EMB, OUT, bm, bn):
    grid = (B // bm, pl.cdiv(OUT, bn))
    return pl.pallas_call(
        _mm_body,
        grid=grid,
        in_specs=[
            pl.BlockSpec((bm, EMB), lambda i, j: (i, 0)),
            pl.BlockSpec((bn, EMB), lambda i, j: (j, 0)),
            pl.BlockSpec((1, bn), lambda i, j: (0, j)),
        ],
        out_specs=pl.BlockSpec((bm, bn), lambda i, j: (i, j)),
        out_shape=jax.ShapeDtypeStruct((B, OUT), jnp.float32),
        compiler_params=pltpu.CompilerParams(
            dimension_semantics=("parallel", "parallel"),
        ),
    )


def kernel(text_batch, table, W, b):
    B, L = text_batch.shape
    V, EMB = table.shape
    OUT = W.shape[0]
    pooled = _make_sc_pool(B, L, EMB, V)(text_batch.astype(jnp.int32), table)
    logits = _make_tc_matmul(B, EMB, OUT, 1024, 2048)(pooled, W, b.reshape(1, OUT))
    return logits
